# baseline (speedup 1.0000x reference)
import functools

import jax
import jax.numpy as jnp
from jax import lax
from jax.experimental import pallas as pl
from jax.experimental.pallas import tpu as pltpu
from jax.experimental.pallas import tpu_sc as plsc

_N = 10000
_E = 320000
_D = 128
_L = 3
_TEMP = 0.01
_NC, _NS = 2, 16
_CH = 128
_K0, _K1 = 48, 112
_KM = _K1
_EP = _NS * (_K0 + _K1) * _CH
_NP = 10240
_RPT = _NP // _NS
_RZ = 128
_BN = 1000
_GRID = _N // _BN


def _make_spmm(d_feat, name):
    mesh = plsc.VectorSubcoreMesh(core_axis_name="c", subcore_axis_name="s")

    @functools.partial(
        pl.kernel,
        out_type=[jax.ShapeDtypeStruct((_NC, _NP, d_feat), jnp.float32),
                  jax.ShapeDtypeStruct((_NC, 2, _NP), jnp.float32)],
        mesh=mesh,
        scratch_types=[
            pltpu.VMEM((_KM, _CH), jnp.int32),
            pltpu.VMEM((_KM, _CH), jnp.int32),
            pltpu.VMEM((_RZ, d_feat), jnp.float32),
            pltpu.VMEM((_CH,), jnp.float32),
            pltpu.VMEM((_CH,), jnp.float32),
            pltpu.VMEM((_CH,), jnp.float32),
            pltpu.VMEM((_CH,), jnp.float32),
            pltpu.VMEM((_RPT,), jnp.float32),
            pltpu.VMEM_SHARED((_NP, d_feat), jnp.float32),
            pltpu.VMEM_SHARED((_NP,), jnp.float32),
            pltpu.VMEM_SHARED((_NP,), jnp.float32),
            pltpu.SemaphoreType.DMA,
            pltpu.SemaphoreType.DMA,
            pltpu.SemaphoreType.DMA,
            pltpu.SemaphoreType.DMA,
            pltpu.SemaphoreType.DMA,
        ],
        name=name,
    )
    def spmm(table_hbm, tb_hbm, ta_hbm, zeros_hbm, zeros1_hbm,
             ui0_hbm, vi0_hbm, ui1_hbm, vi1_hbm, out_hbm, out2_hbm,
             uvm, vvm, gbufa, wca, wra, wcb, wrb, zbuf,
             accum, accc, accr, gsem, wgsema, wgsemb, wssema, wssemb):
        c = lax.axis_index("c")
        s = lax.axis_index("s")
        pltpu.sync_copy(zeros_hbm, gbufa)
        for z in range(_RPT // _RZ):
            pltpu.sync_copy(gbufa, accum.at[pl.ds(s * _RPT + z * _RZ, _RZ), :])
        pltpu.sync_copy(zeros1_hbm, zbuf)
        pltpu.sync_copy(zbuf, accc.at[pl.ds(s * _RPT, _RPT)])
        pltpu.sync_copy(zbuf, accr.at[pl.ds(s * _RPT, _RPT)])

        @pl.when(c == 0)
        def _():
            pltpu.sync_copy(ui0_hbm.at[s], uvm.at[pl.ds(0, _K0)])
            pltpu.sync_copy(vi0_hbm.at[s], vvm.at[pl.ds(0, _K0)])

        @pl.when(c == 1)
        def _():
            pltpu.sync_copy(ui1_hbm.at[s], uvm)
            pltpu.sync_copy(vi1_hbm.at[s], vvm)

        plsc.subcore_barrier()

        def half(i, k, wc, wr, wgsem, wssem):
            @pl.when(i > 0)
            def _():
                pltpu.make_async_copy(wc, accc.at[vvm.at[k]], wssem).wait()
                pltpu.make_async_copy(wr, accr.at[uvm.at[k]], wssem).wait()

            pltpu.async_copy(tb_hbm.at[uvm.at[k]], wc, wgsem)
            pltpu.async_copy(ta_hbm.at[vvm.at[k]], wr, wgsem)
            pltpu.async_copy(table_hbm.at[uvm.at[k]], gbufa, gsem).wait()
            pltpu.sync_copy(gbufa, accum.at[vvm.at[k]], add=True)
            pltpu.make_async_copy(tb_hbm.at[uvm.at[k]], wc, wgsem).wait()
            pltpu.make_async_copy(ta_hbm.at[vvm.at[k]], wr, wgsem).wait()
            pltpu.async_copy(wc, accc.at[vvm.at[k]], wssem, add=True)
            pltpu.async_copy(wr, accr.at[uvm.at[k]], wssem, add=True)

        def body(i, carry):
            half(i, 2 * i, wca, wra, wgsema, wssema)
            half(i, 2 * i + 1, wcb, wrb, wgsemb, wssemb)
            return carry

        lax.fori_loop(0, jnp.where(c == 0, _K0 // 2, _K1 // 2), body, 0)
        pltpu.make_async_copy(wca, accc.at[vvm.at[0]], wssema).wait()
        pltpu.make_async_copy(wra, accr.at[uvm.at[0]], wssema).wait()
        pltpu.make_async_copy(wcb, accc.at[vvm.at[0]], wssemb).wait()
        pltpu.make_async_copy(wrb, accr.at[uvm.at[0]], wssemb).wait()
        plsc.subcore_barrier()
        pltpu.sync_copy(accum.at[pl.ds(s * _RPT, _RPT), :],
                        out_hbm.at[c, pl.ds(s * _RPT, _RPT), :])
        pltpu.sync_copy(accc.at[pl.ds(s * _RPT, _RPT)],
                        out2_hbm.at[c, 0, pl.ds(s * _RPT, _RPT)])
        pltpu.sync_copy(accr.at[pl.ds(s * _RPT, _RPT)],
                        out2_hbm.at[c, 1, pl.ds(s * _RPT, _RPT)])

    return spmm


_spmm128 = _make_spmm(_D, "spmm128")


def _ln_block(h, g, b):
    mu = jnp.mean(h, axis=-1, keepdims=True)
    var = jnp.mean((h - mu) ** 2, axis=-1, keepdims=True)
    return (h - mu) / jnp.sqrt(var + 1e-5) * g + b


def _enc_body(x_ref, we_ref, be_ref, o_ref):
    o_ref[...] = jax.nn.relu(jnp.dot(x_ref[...], we_ref[...]) + be_ref[...])


def _stage1_body(h_ref, g_ref, b_ref, hn_ref):
    hn_ref[...] = _ln_block(h_ref[...], g_ref[...], b_ref[...])


def _stage2_body(hn_ref, m0_ref, m1_ref, deg_ref, g4_ref, wa4_ref, wr4_ref,
                 b4_ref, zb_ref, av_ref, bv_ref):
    hn = hn_ref[...]
    mean = (m0_ref[...] + m1_ref[...]) / jnp.clip(deg_ref[...], 1.0, None)
    logits = jnp.dot(hn, wr4_ref[...]) + jnp.dot(mean, wa4_ref[...]) \
        + b4_ref[...]
    s4 = (logits + g4_ref[...]) / jnp.float32(_TEMP)
    a = (s4[:, 0:1] >= s4[:, 1:2]).astype(jnp.float32)
    b = (s4[:, 2:3] >= s4[:, 3:4]).astype(jnp.float32)
    zb_ref[...] = hn * b
    av_ref[...] = a
    bv_ref[...] = b


def _stage3_body(hn_ref, s0_ref, s1_ref, cnt_ref, rev_ref, av_ref, bv_ref,
                 wr_ref, wa_ref, be_ref, h_ref, st_ref):
    hn = hn_ref[...]
    s = s0_ref[...] + s1_ref[...]
    cnt = cnt_ref[...]
    rev = rev_ref[...]
    a = av_ref[...]
    b = bv_ref[...]
    mean = a * s / jnp.clip(cnt, 1.0, None)
    out = jnp.dot(hn, wr_ref[...]) + jnp.dot(mean, wa_ref[...])
    out = jax.nn.relu(out + be_ref[...])
    h_ref[...] = hn + out
    is_l = (a > 0.5) & (cnt > 0.5)
    is_b = (b > 0.5) & (rev > 0.5)
    st_ref[...] = jnp.where(
        is_b & is_l, 0, jnp.where(is_l, 2, jnp.where(is_b, 1, 3))
    ).astype(jnp.int32)


def _dec_body(h_ref, g_ref, b_ref, wd_ref, bd_ref, o_ref):
    hn = _ln_block(h_ref[...], g_ref[...], b_ref[...])
    o_ref[...] = jnp.dot(hn, wd_ref[...]) + bd_ref[...]


def _row_spec(w):
    return pl.BlockSpec((_BN, w), lambda i: (i, 0))


def _full_spec(r, c):
    return pl.BlockSpec((r, c), lambda i: (0, 0))


def _tc_call(body, in_specs, out_specs, out_shapes, args):
    return pl.pallas_call(
        body,
        grid=(_GRID,),
        in_specs=in_specs,
        out_specs=out_specs,
        out_shape=out_shapes,
        compiler_params=pltpu.CompilerParams(
            dimension_semantics=("arbitrary",)),
    )(*args)


def kernel(x, edge_index, W_enc, b_enc, W_root, W_agg, b_env, Win_root,
           Win_agg, b_in, Wout_root, Wout_agg, b_out_a, ln_g, ln_b, W_dec,
           b_dec):
    f32 = jnp.float32
    u = edge_index[0]
    v = edge_index[1]
    pad = _EP - _E
    e0 = _NS * _K0 * _CH
    uflat = jnp.concatenate([u, jnp.zeros((pad,), jnp.int32)])
    vflat = jnp.concatenate([v, jnp.full((pad,), _N, jnp.int32)])
    ui0 = uflat[:e0].reshape(_NS, _K0, _CH)
    vi0 = vflat[:e0].reshape(_NS, _K0, _CH)
    ui1 = uflat[e0:].reshape(_NS, _K1, _CH)
    vi1 = vflat[e0:].reshape(_NS, _K1, _CH)
    zeros128 = jnp.zeros((_RZ, _D), f32)
    zeros1 = jnp.zeros((_RPT,), f32)
    onesp = jnp.zeros((_NP,), f32).at[:_N].set(1.0)

    gkey = jax.random.key(42)
    g4s = []
    for l in range(_L):
        gi = jax.random.uniform(jax.random.fold_in(gkey, 2 * l), (_N, 2),
                                minval=1e-6, maxval=1 - 1e-6)
        go = jax.random.uniform(jax.random.fold_in(gkey, 2 * l + 1), (_N, 2),
                                minval=1e-6, maxval=1 - 1e-6)
        g4s.append(jnp.concatenate([-jnp.log(-jnp.log(gi)),
                                    -jnp.log(-jnp.log(go))], axis=1))

    wa4 = jnp.concatenate([Win_agg, Wout_agg], axis=1)
    wr4 = jnp.concatenate([Win_root, Wout_root], axis=1)
    bias4 = jnp.concatenate([b_in, b_out_a]).reshape(1, 4)
    ln_g2 = ln_g.reshape(1, _D)
    ln_b2 = ln_b.reshape(1, _D)
    b_dec2 = b_dec.reshape(1, -1)

    h = _tc_call(
        _enc_body,
        [_row_spec(_D), _full_spec(_D, _D), _full_spec(1, _D)],
        _row_spec(_D),
        jax.ShapeDtypeStruct((_N, _D), f32),
        (x, W_enc, b_enc.reshape(1, _D)),
    )

    deg = None
    states = []
    for l in range(_L):
        hn = _tc_call(
            _stage1_body,
            [_row_spec(_D), _full_spec(1, _D), _full_spec(1, _D)],
            _row_spec(_D),
            jax.ShapeDtypeStruct((_N, _D), f32),
            (h, ln_g2, ln_b2),
        )
        m, maux = _spmm128(hn, onesp, onesp, zeros128, zeros1,
                           ui0, vi0, ui1, vi1)
        if deg is None:
            deg = (maux[0, 0, :_N] + maux[1, 0, :_N]).reshape(_N, 1)
        zb, av, bv = _tc_call(
            _stage2_body,
            [_row_spec(_D), _row_spec(_D), _row_spec(_D), _row_spec(1),
             _row_spec(4), _full_spec(_D, 4), _full_spec(_D, 4),
             _full_spec(1, 4)],
            [_row_spec(_D), _row_spec(1), _row_spec(1)],
            [jax.ShapeDtypeStruct((_N, _D), f32),
             jax.ShapeDtypeStruct((_N, 1), f32),
             jax.ShapeDtypeStruct((_N, 1), f32)],
            (hn, m[0, :_N], m[1, :_N], deg, g4s[l], wa4, wr4, bias4),
        )
        b1p = jnp.pad(bv.reshape(_N), (0, _NP - _N))
        a1p = jnp.pad(av.reshape(_N), (0, _NP - _N))
        s, saux = _spmm128(zb, b1p, a1p, zeros128, zeros1,
                           ui0, vi0, ui1, vi1)
        cnt1 = (saux[0, 0, :_N] + saux[1, 0, :_N]).reshape(_N, 1)
        rev1 = (saux[0, 1, :_N] + saux[1, 1, :_N]).reshape(_N, 1)
        h, st = _tc_call(
            _stage3_body,
            [_row_spec(_D), _row_spec(_D), _row_spec(_D), _row_spec(1),
             _row_spec(1), _row_spec(1), _row_spec(1),
             _full_spec(_D, _D), _full_spec(_D, _D), _full_spec(1, _D)],
            [_row_spec(_D), _row_spec(1)],
            [jax.ShapeDtypeStruct((_N, _D), f32),
             jax.ShapeDtypeStruct((_N, 1), jnp.int32)],
            (hn, s[0, :_N], s[1, :_N], cnt1, rev1, av, bv, W_root[l],
             W_agg[l], b_env[l].reshape(1, _D)),
        )
        states.append(st.reshape(_N))

    n_cls = W_dec.shape[1]
    result = _tc_call(
        _dec_body,
        [_row_spec(_D), _full_spec(1, _D), _full_spec(1, _D),
         _full_spec(_D, n_cls), _full_spec(1, n_cls)],
        _row_spec(n_cls),
        jax.ShapeDtypeStruct((_N, n_cls), f32),
        (h, ln_g2, ln_b2, W_dec, b_dec2),
    )
    return (result, jnp.stack(states))


# speedup vs baseline: 1.2933x; 1.2933x over previous
import functools

import jax
import jax.numpy as jnp
from jax import lax
from jax.experimental import pallas as pl
from jax.experimental.pallas import tpu as pltpu
from jax.experimental.pallas import tpu_sc as plsc

_N = 10000
_E = 320000
_D = 128
_L = 3
_TEMP = 0.01
_NC, _NS = 2, 16
_CH = 128
_K0, _K1 = 112, 48
_KM = max(_K0, _K1)
_EP = _NS * (_K0 + _K1) * _CH
_NP = 10240
_RPT = _NP // _NS
_RZ = 128
_BN = 1000
_GRID = _N // _BN


def _make_spmm(d_feat, name):
    mesh = plsc.VectorSubcoreMesh(core_axis_name="c", subcore_axis_name="s")

    @functools.partial(
        pl.kernel,
        out_type=[jax.ShapeDtypeStruct((_NC, _NP, d_feat), jnp.float32),
                  jax.ShapeDtypeStruct((_NC, 2, _NP), jnp.float32)],
        mesh=mesh,
        scratch_types=[
            pltpu.VMEM((_KM, _CH), jnp.int32),
            pltpu.VMEM((_KM, _CH), jnp.int32),
            pltpu.VMEM((_RZ, d_feat), jnp.float32),
            pltpu.VMEM((_CH,), jnp.float32),
            pltpu.VMEM((_CH,), jnp.float32),
            pltpu.VMEM((_CH,), jnp.float32),
            pltpu.VMEM((_CH,), jnp.float32),
            pltpu.VMEM((_RPT,), jnp.float32),
            pltpu.VMEM_SHARED((_NP, d_feat), jnp.float32),
            pltpu.VMEM_SHARED((_NP,), jnp.float32),
            pltpu.VMEM_SHARED((_NP,), jnp.float32),
            pltpu.SemaphoreType.DMA,
            pltpu.SemaphoreType.DMA,
            pltpu.SemaphoreType.DMA,
            pltpu.SemaphoreType.DMA,
            pltpu.SemaphoreType.DMA,
        ],
        name=name,
    )
    def spmm(table_hbm, tb_hbm, ta_hbm, zeros_hbm, zeros1_hbm,
             ui0_hbm, vi0_hbm, ui1_hbm, vi1_hbm, out_hbm, out2_hbm,
             uvm, vvm, gbufa, wca, wra, wcb, wrb, zbuf,
             accum, accc, accr, gsem, wgsema, wgsemb, wssema, wssemb):
        c = lax.axis_index("c")
        s = lax.axis_index("s")
        pltpu.sync_copy(zeros_hbm, gbufa)
        for z in range(_RPT // _RZ):
            pltpu.sync_copy(gbufa, accum.at[pl.ds(s * _RPT + z * _RZ, _RZ), :])
        pltpu.sync_copy(zeros1_hbm, zbuf)
        pltpu.sync_copy(zbuf, accc.at[pl.ds(s * _RPT, _RPT)])
        pltpu.sync_copy(zbuf, accr.at[pl.ds(s * _RPT, _RPT)])

        @pl.when(c == 0)
        def _():
            pltpu.sync_copy(ui0_hbm.at[s], uvm.at[pl.ds(0, _K0)])
            pltpu.sync_copy(vi0_hbm.at[s], vvm.at[pl.ds(0, _K0)])

        @pl.when(c == 1)
        def _():
            pltpu.sync_copy(ui1_hbm.at[s], uvm.at[pl.ds(0, _K1)])
            pltpu.sync_copy(vi1_hbm.at[s], vvm.at[pl.ds(0, _K1)])

        plsc.subcore_barrier()

        def half(i, k, wc, wr, wgsem, wssem):
            @pl.when(i > 0)
            def _():
                pltpu.make_async_copy(wc, accc.at[vvm.at[k]], wssem).wait()
                pltpu.make_async_copy(wr, accr.at[uvm.at[k]], wssem).wait()

            pltpu.async_copy(tb_hbm.at[uvm.at[k]], wc, wgsem)
            pltpu.async_copy(ta_hbm.at[vvm.at[k]], wr, wgsem)
            pltpu.async_copy(table_hbm.at[uvm.at[k]], gbufa, gsem).wait()
            pltpu.sync_copy(gbufa, accum.at[vvm.at[k]], add=True)
            pltpu.make_async_copy(tb_hbm.at[uvm.at[k]], wc, wgsem).wait()
            pltpu.make_async_copy(ta_hbm.at[vvm.at[k]], wr, wgsem).wait()
            pltpu.async_copy(wc, accc.at[vvm.at[k]], wssem, add=True)
            pltpu.async_copy(wr, accr.at[uvm.at[k]], wssem, add=True)

        def body(i, carry):
            half(i, 2 * i, wca, wra, wgsema, wssema)
            half(i, 2 * i + 1, wcb, wrb, wgsemb, wssemb)
            return carry

        lax.fori_loop(0, jnp.where(c == 0, _K0 // 2, _K1 // 2), body, 0)
        pltpu.make_async_copy(wca, accc.at[vvm.at[0]], wssema).wait()
        pltpu.make_async_copy(wra, accr.at[uvm.at[0]], wssema).wait()
        pltpu.make_async_copy(wcb, accc.at[vvm.at[0]], wssemb).wait()
        pltpu.make_async_copy(wrb, accr.at[uvm.at[0]], wssemb).wait()
        plsc.subcore_barrier()
        pltpu.sync_copy(accum.at[pl.ds(s * _RPT, _RPT), :],
                        out_hbm.at[c, pl.ds(s * _RPT, _RPT), :])
        pltpu.sync_copy(accc.at[pl.ds(s * _RPT, _RPT)],
                        out2_hbm.at[c, 0, pl.ds(s * _RPT, _RPT)])
        pltpu.sync_copy(accr.at[pl.ds(s * _RPT, _RPT)],
                        out2_hbm.at[c, 1, pl.ds(s * _RPT, _RPT)])

    return spmm


_spmm128 = _make_spmm(_D, "spmm128")


def _ln_block(h, g, b):
    mu = jnp.mean(h, axis=-1, keepdims=True)
    var = jnp.mean((h - mu) ** 2, axis=-1, keepdims=True)
    return (h - mu) / jnp.sqrt(var + 1e-5) * g + b


def _enc_body(x_ref, we_ref, be_ref, o_ref):
    o_ref[...] = jax.nn.relu(jnp.dot(x_ref[...], we_ref[...]) + be_ref[...])


def _stage1_body(h_ref, g_ref, b_ref, hn_ref):
    hn_ref[...] = _ln_block(h_ref[...], g_ref[...], b_ref[...])


def _stage2_body(hn_ref, m0_ref, m1_ref, deg_ref, g4_ref, wa4_ref, wr4_ref,
                 b4_ref, zb_ref, av_ref, bv_ref):
    hn = hn_ref[...]
    mean = (m0_ref[...] + m1_ref[...]) / jnp.clip(deg_ref[...], 1.0, None)
    logits = jnp.dot(hn, wr4_ref[...]) + jnp.dot(mean, wa4_ref[...]) \
        + b4_ref[...]
    s4 = (logits + g4_ref[...]) / jnp.float32(_TEMP)
    a = (s4[:, 0:1] >= s4[:, 1:2]).astype(jnp.float32)
    b = (s4[:, 2:3] >= s4[:, 3:4]).astype(jnp.float32)
    zb_ref[...] = hn * b
    av_ref[...] = a
    bv_ref[...] = b


def _stage3_body(hn_ref, s0_ref, s1_ref, cnt_ref, rev_ref, av_ref, bv_ref,
                 wr_ref, wa_ref, be_ref, h_ref, st_ref):
    hn = hn_ref[...]
    s = s0_ref[...] + s1_ref[...]
    cnt = cnt_ref[...]
    rev = rev_ref[...]
    a = av_ref[...]
    b = bv_ref[...]
    mean = a * s / jnp.clip(cnt, 1.0, None)
    out = jnp.dot(hn, wr_ref[...]) + jnp.dot(mean, wa_ref[...])
    out = jax.nn.relu(out + be_ref[...])
    h_ref[...] = hn + out
    is_l = (a > 0.5) & (cnt > 0.5)
    is_b = (b > 0.5) & (rev > 0.5)
    st_ref[...] = jnp.where(
        is_b & is_l, 0, jnp.where(is_l, 2, jnp.where(is_b, 1, 3))
    ).astype(jnp.int32)


def _dec_body(h_ref, g_ref, b_ref, wd_ref, bd_ref, o_ref):
    hn = _ln_block(h_ref[...], g_ref[...], b_ref[...])
    o_ref[...] = jnp.dot(hn, wd_ref[...]) + bd_ref[...]


def _row_spec(w):
    return pl.BlockSpec((_BN, w), lambda i: (i, 0))


def _full_spec(r, c):
    return pl.BlockSpec((r, c), lambda i: (0, 0))


def _tc_call(body, in_specs, out_specs, out_shapes, args):
    return pl.pallas_call(
        body,
        grid=(_GRID,),
        in_specs=in_specs,
        out_specs=out_specs,
        out_shape=out_shapes,
        compiler_params=pltpu.CompilerParams(
            dimension_semantics=("arbitrary",)),
    )(*args)


def kernel(x, edge_index, W_enc, b_enc, W_root, W_agg, b_env, Win_root,
           Win_agg, b_in, Wout_root, Wout_agg, b_out_a, ln_g, ln_b, W_dec,
           b_dec):
    f32 = jnp.float32
    u = edge_index[0]
    v = edge_index[1]
    pad = _EP - _E
    e0 = _NS * _K0 * _CH
    uflat = jnp.concatenate([u, jnp.zeros((pad,), jnp.int32)])
    vflat = jnp.concatenate([v, jnp.full((pad,), _N, jnp.int32)])
    ui0 = uflat[:e0].reshape(_NS, _K0, _CH)
    vi0 = vflat[:e0].reshape(_NS, _K0, _CH)
    ui1 = uflat[e0:].reshape(_NS, _K1, _CH)
    vi1 = vflat[e0:].reshape(_NS, _K1, _CH)
    zeros128 = jnp.zeros((_RZ, _D), f32)
    zeros1 = jnp.zeros((_RPT,), f32)
    onesp = jnp.zeros((_NP,), f32).at[:_N].set(1.0)

    gkey = jax.random.key(42)
    g4s = []
    for l in range(_L):
        gi = jax.random.uniform(jax.random.fold_in(gkey, 2 * l), (_N, 2),
                                minval=1e-6, maxval=1 - 1e-6)
        go = jax.random.uniform(jax.random.fold_in(gkey, 2 * l + 1), (_N, 2),
                                minval=1e-6, maxval=1 - 1e-6)
        g4s.append(jnp.concatenate([-jnp.log(-jnp.log(gi)),
                                    -jnp.log(-jnp.log(go))], axis=1))

    wa4 = jnp.concatenate([Win_agg, Wout_agg], axis=1)
    wr4 = jnp.concatenate([Win_root, Wout_root], axis=1)
    bias4 = jnp.concatenate([b_in, b_out_a]).reshape(1, 4)
    ln_g2 = ln_g.reshape(1, _D)
    ln_b2 = ln_b.reshape(1, _D)
    b_dec2 = b_dec.reshape(1, -1)

    h = _tc_call(
        _enc_body,
        [_row_spec(_D), _full_spec(_D, _D), _full_spec(1, _D)],
        _row_spec(_D),
        jax.ShapeDtypeStruct((_N, _D), f32),
        (x, W_enc, b_enc.reshape(1, _D)),
    )

    deg = None
    states = []
    for l in range(_L):
        hn = _tc_call(
            _stage1_body,
            [_row_spec(_D), _full_spec(1, _D), _full_spec(1, _D)],
            _row_spec(_D),
            jax.ShapeDtypeStruct((_N, _D), f32),
            (h, ln_g2, ln_b2),
        )
        m, maux = _spmm128(hn, onesp, onesp, zeros128, zeros1,
                           ui0, vi0, ui1, vi1)
        if deg is None:
            deg = (maux[0, 0, :_N] + maux[1, 0, :_N]).reshape(_N, 1)
        zb, av, bv = _tc_call(
            _stage2_body,
            [_row_spec(_D), _row_spec(_D), _row_spec(_D), _row_spec(1),
             _row_spec(4), _full_spec(_D, 4), _full_spec(_D, 4),
             _full_spec(1, 4)],
            [_row_spec(_D), _row_spec(1), _row_spec(1)],
            [jax.ShapeDtypeStruct((_N, _D), f32),
             jax.ShapeDtypeStruct((_N, 1), f32),
             jax.ShapeDtypeStruct((_N, 1), f32)],
            (hn, m[0, :_N], m[1, :_N], deg, g4s[l], wa4, wr4, bias4),
        )
        b1p = jnp.pad(bv.reshape(_N), (0, _NP - _N))
        a1p = jnp.pad(av.reshape(_N), (0, _NP - _N))
        s, saux = _spmm128(zb, b1p, a1p, zeros128, zeros1,
                           ui0, vi0, ui1, vi1)
        cnt1 = (saux[0, 0, :_N] + saux[1, 0, :_N]).reshape(_N, 1)
        rev1 = (saux[0, 1, :_N] + saux[1, 1, :_N]).reshape(_N, 1)
        h, st = _tc_call(
            _stage3_body,
            [_row_spec(_D), _row_spec(_D), _row_spec(_D), _row_spec(1),
             _row_spec(1), _row_spec(1), _row_spec(1),
             _full_spec(_D, _D), _full_spec(_D, _D), _full_spec(1, _D)],
            [_row_spec(_D), _row_spec(1)],
            [jax.ShapeDtypeStruct((_N, _D), f32),
             jax.ShapeDtypeStruct((_N, 1), jnp.int32)],
            (hn, s[0, :_N], s[1, :_N], cnt1, rev1, av, bv, W_root[l],
             W_agg[l], b_env[l].reshape(1, _D)),
        )
        states.append(st.reshape(_N))

    n_cls = W_dec.shape[1]
    result = _tc_call(
        _dec_body,
        [_row_spec(_D), _full_spec(1, _D), _full_spec(1, _D),
         _full_spec(_D, n_cls), _full_spec(1, n_cls)],
        _row_spec(n_cls),
        jax.ShapeDtypeStruct((_N, n_cls), f32),
        (h, ln_g2, ln_b2, W_dec, b_dec2),
    )
    return (result, jnp.stack(states))


# speedup vs baseline: 1.4847x; 1.1480x over previous
import functools

import jax
import jax.numpy as jnp
from jax import lax
from jax.experimental import pallas as pl
from jax.experimental.pallas import tpu as pltpu
from jax.experimental.pallas import tpu_sc as plsc

_N = 10000
_E = 320000
_D = 128
_L = 3
_TEMP = 0.01
_NC, _NS = 2, 16
_CH = 128
_K0, _K1 = 128, 32
_KB = 64
_EP = _NS * (_K0 + _K1) * _CH
_NP = 10240
_RPT = _NP // _NS
_RZ = 128
_BN = 1000
_GRID = _N // _BN


def _make_spmm(d_feat, name):
    mesh = plsc.VectorSubcoreMesh(core_axis_name="c", subcore_axis_name="s")

    @functools.partial(
        pl.kernel,
        out_type=[jax.ShapeDtypeStruct((_NC, _NP, d_feat), jnp.float32),
                  jax.ShapeDtypeStruct((_NC, 2, _NP), jnp.float32)],
        mesh=mesh,
        scratch_types=[
            pltpu.VMEM((_KB, _CH), jnp.int32),
            pltpu.VMEM((_KB, _CH), jnp.int32),
            pltpu.VMEM((_RZ, d_feat), jnp.float32),
            pltpu.VMEM((_CH,), jnp.float32),
            pltpu.VMEM((_CH,), jnp.float32),
            pltpu.VMEM((_CH,), jnp.float32),
            pltpu.VMEM((_CH,), jnp.float32),
            pltpu.VMEM((_RPT,), jnp.float32),
            pltpu.VMEM_SHARED((_NP, d_feat), jnp.float32),
            pltpu.VMEM_SHARED((_NP,), jnp.float32),
            pltpu.VMEM_SHARED((_NP,), jnp.float32),
            pltpu.SemaphoreType.DMA,
            pltpu.SemaphoreType.DMA,
            pltpu.SemaphoreType.DMA,
            pltpu.SemaphoreType.DMA,
            pltpu.SemaphoreType.DMA,
        ],
        name=name,
    )
    def spmm(table_hbm, tb_hbm, ta_hbm, zeros_hbm, zeros1_hbm,
             ui0_hbm, vi0_hbm, ui1_hbm, vi1_hbm, out_hbm, out2_hbm,
             uvm, vvm, gbufa, wca, wra, wcb, wrb, zbuf,
             accum, accc, accr, gsem, wgsema, wgsemb, wssema, wssemb):
        c = lax.axis_index("c")
        s = lax.axis_index("s")
        pltpu.sync_copy(zeros_hbm, gbufa)
        for z in range(_RPT // _RZ):
            pltpu.sync_copy(gbufa, accum.at[pl.ds(s * _RPT + z * _RZ, _RZ), :])
        pltpu.sync_copy(zeros1_hbm, zbuf)
        pltpu.sync_copy(zbuf, accc.at[pl.ds(s * _RPT, _RPT)])
        pltpu.sync_copy(zbuf, accr.at[pl.ds(s * _RPT, _RPT)])

        plsc.subcore_barrier()

        def half(i, k, wc, wr, wgsem, wssem):
            @pl.when(i > 0)
            def _():
                pltpu.make_async_copy(wc, accc.at[vvm.at[k]], wssem).wait()
                pltpu.make_async_copy(wr, accr.at[uvm.at[k]], wssem).wait()

            pltpu.async_copy(tb_hbm.at[uvm.at[k]], wc, wgsem)
            pltpu.async_copy(ta_hbm.at[vvm.at[k]], wr, wgsem)
            pltpu.async_copy(table_hbm.at[uvm.at[k]], gbufa, gsem).wait()
            pltpu.sync_copy(gbufa, accum.at[vvm.at[k]], add=True)
            pltpu.make_async_copy(tb_hbm.at[uvm.at[k]], wc, wgsem).wait()
            pltpu.make_async_copy(ta_hbm.at[vvm.at[k]], wr, wgsem).wait()
            pltpu.async_copy(wc, accc.at[vvm.at[k]], wssem, add=True)
            pltpu.async_copy(wr, accr.at[uvm.at[k]], wssem, add=True)

        def body(i, carry):
            half(i, 2 * i, wca, wra, wgsema, wssema)
            half(i, 2 * i + 1, wcb, wrb, wgsemb, wssemb)
            return carry

        def batch(b, carry):
            @pl.when(c == 0)
            def _():
                off = pl.multiple_of(b * _KB, _KB)
                pltpu.sync_copy(ui0_hbm.at[s, pl.ds(off, _KB)], uvm)
                pltpu.sync_copy(vi0_hbm.at[s, pl.ds(off, _KB)], vvm)

            @pl.when(c == 1)
            def _():
                pltpu.sync_copy(ui1_hbm.at[s], uvm.at[pl.ds(0, _K1)])
                pltpu.sync_copy(vi1_hbm.at[s], vvm.at[pl.ds(0, _K1)])

            lax.fori_loop(0, jnp.where(c == 0, _KB // 2, _K1 // 2), body, 0)
            pltpu.make_async_copy(wca, accc.at[vvm.at[0]], wssema).wait()
            pltpu.make_async_copy(wra, accr.at[uvm.at[0]], wssema).wait()
            pltpu.make_async_copy(wcb, accc.at[vvm.at[0]], wssemb).wait()
            pltpu.make_async_copy(wrb, accr.at[uvm.at[0]], wssemb).wait()
            return carry

        lax.fori_loop(0, jnp.where(c == 0, _K0 // _KB, 1), batch, 0)
        plsc.subcore_barrier()
        pltpu.sync_copy(accum.at[pl.ds(s * _RPT, _RPT), :],
                        out_hbm.at[c, pl.ds(s * _RPT, _RPT), :])
        pltpu.sync_copy(accc.at[pl.ds(s * _RPT, _RPT)],
                        out2_hbm.at[c, 0, pl.ds(s * _RPT, _RPT)])
        pltpu.sync_copy(accr.at[pl.ds(s * _RPT, _RPT)],
                        out2_hbm.at[c, 1, pl.ds(s * _RPT, _RPT)])

    return spmm


_spmm128 = _make_spmm(_D, "spmm128")


def _ln_block(h, g, b):
    mu = jnp.mean(h, axis=-1, keepdims=True)
    var = jnp.mean((h - mu) ** 2, axis=-1, keepdims=True)
    return (h - mu) / jnp.sqrt(var + 1e-5) * g + b


def _enc_body(x_ref, we_ref, be_ref, o_ref):
    o_ref[...] = jax.nn.relu(jnp.dot(x_ref[...], we_ref[...]) + be_ref[...])


def _stage1_body(h_ref, g_ref, b_ref, hn_ref):
    hn_ref[...] = _ln_block(h_ref[...], g_ref[...], b_ref[...])


def _stage2_body(hn_ref, m0_ref, m1_ref, deg_ref, g4_ref, wa4_ref, wr4_ref,
                 b4_ref, zb_ref, av_ref, bv_ref):
    hn = hn_ref[...]
    mean = (m0_ref[...] + m1_ref[...]) / jnp.clip(deg_ref[...], 1.0, None)
    logits = jnp.dot(hn, wr4_ref[...]) + jnp.dot(mean, wa4_ref[...]) \
        + b4_ref[...]
    s4 = (logits + g4_ref[...]) / jnp.float32(_TEMP)
    a = (s4[:, 0:1] >= s4[:, 1:2]).astype(jnp.float32)
    b = (s4[:, 2:3] >= s4[:, 3:4]).astype(jnp.float32)
    zb_ref[...] = hn * b
    av_ref[...] = a
    bv_ref[...] = b


def _stage3_body(hn_ref, s0_ref, s1_ref, cnt_ref, rev_ref, av_ref, bv_ref,
                 wr_ref, wa_ref, be_ref, h_ref, st_ref):
    hn = hn_ref[...]
    s = s0_ref[...] + s1_ref[...]
    cnt = cnt_ref[...]
    rev = rev_ref[...]
    a = av_ref[...]
    b = bv_ref[...]
    mean = a * s / jnp.clip(cnt, 1.0, None)
    out = jnp.dot(hn, wr_ref[...]) + jnp.dot(mean, wa_ref[...])
    out = jax.nn.relu(out + be_ref[...])
    h_ref[...] = hn + out
    is_l = (a > 0.5) & (cnt > 0.5)
    is_b = (b > 0.5) & (rev > 0.5)
    st_ref[...] = jnp.where(
        is_b & is_l, 0, jnp.where(is_l, 2, jnp.where(is_b, 1, 3))
    ).astype(jnp.int32)


def _dec_body(h_ref, g_ref, b_ref, wd_ref, bd_ref, o_ref):
    hn = _ln_block(h_ref[...], g_ref[...], b_ref[...])
    o_ref[...] = jnp.dot(hn, wd_ref[...]) + bd_ref[...]


def _row_spec(w):
    return pl.BlockSpec((_BN, w), lambda i: (i, 0))


def _full_spec(r, c):
    return pl.BlockSpec((r, c), lambda i: (0, 0))


def _tc_call(body, in_specs, out_specs, out_shapes, args):
    return pl.pallas_call(
        body,
        grid=(_GRID,),
        in_specs=in_specs,
        out_specs=out_specs,
        out_shape=out_shapes,
        compiler_params=pltpu.CompilerParams(
            dimension_semantics=("arbitrary",)),
    )(*args)


def kernel(x, edge_index, W_enc, b_enc, W_root, W_agg, b_env, Win_root,
           Win_agg, b_in, Wout_root, Wout_agg, b_out_a, ln_g, ln_b, W_dec,
           b_dec):
    f32 = jnp.float32
    u = edge_index[0]
    v = edge_index[1]
    pad = _EP - _E
    e0 = _NS * _K0 * _CH
    uflat = jnp.concatenate([u, jnp.zeros((pad,), jnp.int32)])
    vflat = jnp.concatenate([v, jnp.full((pad,), _N, jnp.int32)])
    ui0 = uflat[:e0].reshape(_NS, _K0, _CH)
    vi0 = vflat[:e0].reshape(_NS, _K0, _CH)
    ui1 = uflat[e0:].reshape(_NS, _K1, _CH)
    vi1 = vflat[e0:].reshape(_NS, _K1, _CH)
    zeros128 = jnp.zeros((_RZ, _D), f32)
    zeros1 = jnp.zeros((_RPT,), f32)
    onesp = jnp.zeros((_NP,), f32).at[:_N].set(1.0)

    gkey = jax.random.key(42)
    g4s = []
    for l in range(_L):
        gi = jax.random.uniform(jax.random.fold_in(gkey, 2 * l), (_N, 2),
                                minval=1e-6, maxval=1 - 1e-6)
        go = jax.random.uniform(jax.random.fold_in(gkey, 2 * l + 1), (_N, 2),
                                minval=1e-6, maxval=1 - 1e-6)
        g4s.append(jnp.concatenate([-jnp.log(-jnp.log(gi)),
                                    -jnp.log(-jnp.log(go))], axis=1))

    wa4 = jnp.concatenate([Win_agg, Wout_agg], axis=1)
    wr4 = jnp.concatenate([Win_root, Wout_root], axis=1)
    bias4 = jnp.concatenate([b_in, b_out_a]).reshape(1, 4)
    ln_g2 = ln_g.reshape(1, _D)
    ln_b2 = ln_b.reshape(1, _D)
    b_dec2 = b_dec.reshape(1, -1)

    h = _tc_call(
        _enc_body,
        [_row_spec(_D), _full_spec(_D, _D), _full_spec(1, _D)],
        _row_spec(_D),
        jax.ShapeDtypeStruct((_N, _D), f32),
        (x, W_enc, b_enc.reshape(1, _D)),
    )

    deg = None
    states = []
    for l in range(_L):
        hn = _tc_call(
            _stage1_body,
            [_row_spec(_D), _full_spec(1, _D), _full_spec(1, _D)],
            _row_spec(_D),
            jax.ShapeDtypeStruct((_N, _D), f32),
            (h, ln_g2, ln_b2),
        )
        m, maux = _spmm128(hn, onesp, onesp, zeros128, zeros1,
                           ui0, vi0, ui1, vi1)
        if deg is None:
            deg = (maux[0, 0, :_N] + maux[1, 0, :_N]).reshape(_N, 1)
        zb, av, bv = _tc_call(
            _stage2_body,
            [_row_spec(_D), _row_spec(_D), _row_spec(_D), _row_spec(1),
             _row_spec(4), _full_spec(_D, 4), _full_spec(_D, 4),
             _full_spec(1, 4)],
            [_row_spec(_D), _row_spec(1), _row_spec(1)],
            [jax.ShapeDtypeStruct((_N, _D), f32),
             jax.ShapeDtypeStruct((_N, 1), f32),
             jax.ShapeDtypeStruct((_N, 1), f32)],
            (hn, m[0, :_N], m[1, :_N], deg, g4s[l], wa4, wr4, bias4),
        )
        b1p = jnp.pad(bv.reshape(_N), (0, _NP - _N))
        a1p = jnp.pad(av.reshape(_N), (0, _NP - _N))
        s, saux = _spmm128(zb, b1p, a1p, zeros128, zeros1,
                           ui0, vi0, ui1, vi1)
        cnt1 = (saux[0, 0, :_N] + saux[1, 0, :_N]).reshape(_N, 1)
        rev1 = (saux[0, 1, :_N] + saux[1, 1, :_N]).reshape(_N, 1)
        h, st = _tc_call(
            _stage3_body,
            [_row_spec(_D), _row_spec(_D), _row_spec(_D), _row_spec(1),
             _row_spec(1), _row_spec(1), _row_spec(1),
             _full_spec(_D, _D), _full_spec(_D, _D), _full_spec(1, _D)],
            [_row_spec(_D), _row_spec(1)],
            [jax.ShapeDtypeStruct((_N, _D), f32),
             jax.ShapeDtypeStruct((_N, 1), jnp.int32)],
            (hn, s[0, :_N], s[1, :_N], cnt1, rev1, av, bv, W_root[l],
             W_agg[l], b_env[l].reshape(1, _D)),
        )
        states.append(st.reshape(_N))

    n_cls = W_dec.shape[1]
    result = _tc_call(
        _dec_body,
        [_row_spec(_D), _full_spec(1, _D), _full_spec(1, _D),
         _full_spec(_D, n_cls), _full_spec(1, n_cls)],
        _row_spec(n_cls),
        jax.ShapeDtypeStruct((_N, n_cls), f32),
        (h, ln_g2, ln_b2, W_dec, b_dec2),
    )
    return (result, jnp.stack(states))
